# baseline (speedup 1.0000x reference)
import functools

import jax
import jax.numpy as jnp
from jax import lax
from jax.experimental import pallas as pl
from jax.experimental.pallas import tpu as pltpu
from jax.experimental.pallas import tpu_sc as plsc

B = 4096
L = 50
D_EMBED = 64
N_FEATURE = 16
D_FEATURE = 32
D_OUT = D_EMBED + D_FEATURE
N_TOKEN = 1000000
TAB_W = 128
N_ROW = N_TOKEN // 2

NC, NS = 2, 16
NW = NC * NS
BPW = B // NW

SLAB_T = 512
SLAB_R = SLAB_T // 2
SPW = 61
EXTRA_T0 = N_TOKEN - 64 - SLAB_T
TAIL_T = 64
TAIL_R = TAIL_T // 2


def _make_relayout_kernel():
    mesh = plsc.VectorSubcoreMesh(core_axis_name="c", subcore_axis_name="s")

    @functools.partial(
        pl.kernel,
        mesh=mesh,
        compiler_params=pltpu.CompilerParams(needs_layout_passes=False),
        out_type=jax.ShapeDtypeStruct((N_ROW, TAB_W), jnp.float32),
        scratch_types=[
            pltpu.VMEM((D_EMBED, SLAB_T), jnp.float32),
            pltpu.VMEM((D_EMBED, SLAB_T), jnp.float32),
            pltpu.VMEM((SLAB_R, TAB_W), jnp.float32),
            pltpu.VMEM((TAIL_T, D_EMBED), jnp.float32),
            pltpu.SemaphoreType.DMA,
            pltpu.SemaphoreType.DMA,
            pltpu.SemaphoreType.DMA,
        ],
    )
    def k(tab_hbm, tail_hbm, out_hbm, srcA, srcB, out_v, tail_v,
          sem_ia, sem_ib, sem_o):
        cid = lax.axis_index("c")
        sid = lax.axis_index("s")
        wid = cid * NS + sid
        slab0 = wid * SPW

        lane = lax.iota(jnp.int32, 16)

        def start_in(t0, buf, sem):
            pltpu.async_copy(tab_hbm.at[:, pl.ds(t0, SLAB_T)], buf, sem)

        def wait_in(buf, sem):
            pltpu.make_async_copy(tab_hbm.at[:, pl.ds(0, SLAB_T)], buf,
                                  sem).wait()

        def compute(src_v):
            def r_body(rq, c2):
                for dr in range(4):
                    r = rq * 4 + dr
                    for h in range(2):
                        tv = jnp.full((16,), r * 2 + h, jnp.int32)
                        for cg in range(4):
                            val = plsc.load_gather(src_v,
                                                   [lane + cg * 16, tv])
                            out_v[r, pl.ds(h * 64 + cg * 16, 16)] = val
                return c2

            lax.fori_loop(0, SLAB_R // 4, r_body, 0)

        def flush_out(r0):
            pltpu.async_copy(out_v, out_hbm.at[pl.ds(r0, SLAB_R)],
                             sem_o).wait()

        start_in(pl.multiple_of(slab0 * SLAB_T, SLAB_T), srcA, sem_ia)

        def pair_body(i, carry):
            sA = slab0 + 2 * i
            start_in(pl.multiple_of((sA + 1) * SLAB_T, SLAB_T), srcB, sem_ib)
            wait_in(srcA, sem_ia)
            compute(srcA)
            flush_out(pl.multiple_of(sA * SLAB_R, SLAB_R))

            @pl.when(i < SPW // 2 - 1)
            def _():
                start_in(pl.multiple_of((sA + 2) * SLAB_T, SLAB_T), srcA,
                         sem_ia)

            wait_in(srcB, sem_ib)
            compute(srcB)
            flush_out(pl.multiple_of((sA + 1) * SLAB_R, SLAB_R))
            return carry

        lax.fori_loop(0, SPW // 2, pair_body, 0)

        s_last = slab0 + SPW - 1
        start_in(pl.multiple_of(s_last * SLAB_T, SLAB_T), srcA, sem_ia)
        wait_in(srcA, sem_ia)
        compute(srcA)
        flush_out(pl.multiple_of(s_last * SLAB_R, SLAB_R))

        @pl.when(wid == 0)
        def _():
            start_in(EXTRA_T0, srcB, sem_ib)
            wait_in(srcB, sem_ib)
            compute(srcB)
            flush_out(EXTRA_T0 // 2)

        @pl.when(wid == NW - 1)
        def _():
            pltpu.sync_copy(tail_hbm, tail_v)
            for rr in range(TAIL_R):
                for h in range(2):
                    for cg in range(4):
                        out_v[rr, pl.ds(h * 64 + cg * 16, 16)] = (
                            tail_v[rr * 2 + h, pl.ds(cg * 16, 16)])
            pltpu.sync_copy(out_v.at[pl.ds(0, TAIL_R)],
                            out_hbm.at[pl.ds(N_ROW - TAIL_R, TAIL_R)])

    return k


def _make_lookup_kernel():
    mesh = plsc.VectorSubcoreMesh(core_axis_name="c", subcore_axis_name="s")

    @functools.partial(
        pl.kernel,
        mesh=mesh,
        compiler_params=pltpu.CompilerParams(needs_layout_passes=False),
        out_type=jax.ShapeDtypeStruct((L, D_OUT, B), jnp.float32),
        scratch_types=[
            pltpu.VMEM((L, BPW), jnp.int32),
            pltpu.VMEM((BPW,), jnp.int32),
            pltpu.VMEM((BPW,), jnp.int32),
            pltpu.VMEM((BPW, TAB_W), jnp.float32),
            pltpu.VMEM((N_FEATURE, BPW), jnp.float32),
            pltpu.VMEM((D_OUT, BPW), jnp.float32),
            pltpu.VMEM((24, 128), jnp.float32),
            pltpu.SMEM((N_FEATURE * D_FEATURE,), jnp.float32),
            pltpu.SMEM((D_FEATURE,), jnp.float32),
            pltpu.SemaphoreType.DMA,
        ],
    )
    def k(tok_hbm, feat_hbm, table_hbm, aux_hbm, out_hbm,
          tok_v, half_v, par_v, emb_v, feat_v, tile_v, aux_v,
          w_sm, b_sm, sem):
        cid = lax.axis_index("c")
        sid = lax.axis_index("s")
        wid = cid * NS + sid
        boff = pl.multiple_of(wid * BPW, BPW)

        pltpu.sync_copy(tok_hbm.at[:, pl.ds(boff, BPW)], tok_v)
        pltpu.sync_copy(aux_hbm, aux_v)
        for kf in range(N_FEATURE):
            for jh in range(D_FEATURE // 16):
                wv = aux_v[kf, pl.ds(jh * 16, 16)]
                for i in range(16):
                    w_sm[kf * D_FEATURE + jh * 16 + i] = wv[i]
        for jh in range(D_FEATURE // 16):
            bv = aux_v[N_FEATURE, pl.ds(jh * 16, 16)]
            for i in range(16):
                b_sm[jh * 16 + i] = bv[i]

        lane = lax.iota(jnp.int32, 16)

        def l_body(l, carry):
            for g in range(BPW // 16):
                t16 = tok_v[l, pl.ds(g * 16, 16)]
                half_v[pl.ds(g * 16, 16)] = t16 >> 1
                par_v[pl.ds(g * 16, 16)] = (t16 & 1) * D_EMBED
            pltpu.async_copy(table_hbm.at[half_v], emb_v, sem).wait()
            pltpu.sync_copy(feat_hbm.at[l, :, pl.ds(boff, BPW)], feat_v)

            for g in range(BPW // 16):
                fv = [feat_v[kf, pl.ds(g * 16, 16)] for kf in range(N_FEATURE)]

                def j_body(j, c2):
                    acc = fv[0] * w_sm[j] + b_sm[j]
                    for kf in range(1, N_FEATURE):
                        acc = acc + fv[kf] * w_sm[kf * D_FEATURE + j]
                    tile_v[D_EMBED + j, pl.ds(g * 16, 16)] = acc
                    return c2

                lax.fori_loop(0, D_FEATURE, j_body, 0)

            for g in range(BPW // 16):
                rows = lane + (g * 16)
                cols0 = par_v[pl.ds(g * 16, 16)]

                def c_body(c8, c2):
                    for dc in range(8):
                        c = c8 * 8 + dc
                        val = plsc.load_gather(emb_v, [rows, cols0 + c])
                        tile_v[c, pl.ds(g * 16, 16)] = val
                    return c2

                lax.fori_loop(0, D_EMBED // 8, c_body, 0)

            pltpu.sync_copy(tile_v, out_hbm.at[l, :, pl.ds(boff, BPW)])
            return carry

        lax.fori_loop(0, L, l_body, 0)

    return k


_relayout = _make_relayout_kernel()
_lookup = _make_lookup_kernel()


def kernel(tokens, features, embed_table, proj_W, proj_b):
    tok_t = tokens.astype(jnp.int32).T
    feat_t = features.transpose(1, 2, 0)
    tab_t = embed_table.T
    tail = embed_table[N_TOKEN - TAIL_T:, :]
    aux = jnp.zeros((24, 128), jnp.float32)
    aux = aux.at[:N_FEATURE, :D_FEATURE].set(proj_W.T)
    aux = aux.at[N_FEATURE, :D_FEATURE].set(proj_b)
    table2 = _relayout(tab_t, tail)
    out = _lookup(tok_t, feat_t, table2, aux)
    return out.transpose(2, 0, 1)


# speedup vs baseline: 1.0003x; 1.0003x over previous
import functools

import jax
import jax.numpy as jnp
from jax import lax
from jax.experimental import pallas as pl
from jax.experimental.pallas import tpu as pltpu
from jax.experimental.pallas import tpu_sc as plsc

B = 4096
L = 50
D_EMBED = 64
N_FEATURE = 16
D_FEATURE = 32
D_OUT = D_EMBED + D_FEATURE
N_TOKEN = 1000000
TAB_W = 128
N_ROW = N_TOKEN // 2

NC, NS = 2, 16
NW = NC * NS
BPW = B // NW

SLAB_T = 512
SLAB_R = SLAB_T // 2
SPW = 61
EXTRA_T0 = N_TOKEN - 64 - SLAB_T
TAIL_T = 64
TAIL_R = TAIL_T // 2


def _make_relayout_kernel():
    mesh = plsc.VectorSubcoreMesh(core_axis_name="c", subcore_axis_name="s")

    @functools.partial(
        pl.kernel,
        mesh=mesh,
        compiler_params=pltpu.CompilerParams(needs_layout_passes=False, disable_bounds_checks=True),
        out_type=jax.ShapeDtypeStruct((N_ROW, TAB_W), jnp.float32),
        scratch_types=[
            pltpu.VMEM((D_EMBED, SLAB_T), jnp.float32),
            pltpu.VMEM((D_EMBED, SLAB_T), jnp.float32),
            pltpu.VMEM((SLAB_R, TAB_W), jnp.float32),
            pltpu.VMEM((TAIL_T, D_EMBED), jnp.float32),
            pltpu.SemaphoreType.DMA,
            pltpu.SemaphoreType.DMA,
            pltpu.SemaphoreType.DMA,
        ],
    )
    def k(tab_hbm, tail_hbm, out_hbm, srcA, srcB, out_v, tail_v,
          sem_ia, sem_ib, sem_o):
        cid = lax.axis_index("c")
        sid = lax.axis_index("s")
        wid = cid * NS + sid
        slab0 = wid * SPW

        lane = lax.iota(jnp.int32, 16)

        def start_in(t0, buf, sem):
            pltpu.async_copy(tab_hbm.at[:, pl.ds(t0, SLAB_T)], buf, sem)

        def wait_in(buf, sem):
            pltpu.make_async_copy(tab_hbm.at[:, pl.ds(0, SLAB_T)], buf,
                                  sem).wait()

        def compute(src_v):
            def r_body(rq, c2):
                for dr in range(4):
                    r = rq * 4 + dr
                    for h in range(2):
                        tv = jnp.full((16,), r * 2 + h, jnp.int32)
                        for cg in range(4):
                            val = plsc.load_gather(src_v,
                                                   [lane + cg * 16, tv])
                            out_v[r, pl.ds(h * 64 + cg * 16, 16)] = val
                return c2

            lax.fori_loop(0, SLAB_R // 4, r_body, 0)

        def flush_out(r0):
            pltpu.async_copy(out_v, out_hbm.at[pl.ds(r0, SLAB_R)],
                             sem_o).wait()

        start_in(pl.multiple_of(slab0 * SLAB_T, SLAB_T), srcA, sem_ia)

        def pair_body(i, carry):
            sA = slab0 + 2 * i
            start_in(pl.multiple_of((sA + 1) * SLAB_T, SLAB_T), srcB, sem_ib)
            wait_in(srcA, sem_ia)
            compute(srcA)
            flush_out(pl.multiple_of(sA * SLAB_R, SLAB_R))

            @pl.when(i < SPW // 2 - 1)
            def _():
                start_in(pl.multiple_of((sA + 2) * SLAB_T, SLAB_T), srcA,
                         sem_ia)

            wait_in(srcB, sem_ib)
            compute(srcB)
            flush_out(pl.multiple_of((sA + 1) * SLAB_R, SLAB_R))
            return carry

        lax.fori_loop(0, SPW // 2, pair_body, 0)

        s_last = slab0 + SPW - 1
        start_in(pl.multiple_of(s_last * SLAB_T, SLAB_T), srcA, sem_ia)
        wait_in(srcA, sem_ia)
        compute(srcA)
        flush_out(pl.multiple_of(s_last * SLAB_R, SLAB_R))

        @pl.when(wid == 0)
        def _():
            start_in(EXTRA_T0, srcB, sem_ib)
            wait_in(srcB, sem_ib)
            compute(srcB)
            flush_out(EXTRA_T0 // 2)

        @pl.when(wid == NW - 1)
        def _():
            pltpu.sync_copy(tail_hbm, tail_v)
            for rr in range(TAIL_R):
                for h in range(2):
                    for cg in range(4):
                        out_v[rr, pl.ds(h * 64 + cg * 16, 16)] = (
                            tail_v[rr * 2 + h, pl.ds(cg * 16, 16)])
            pltpu.sync_copy(out_v.at[pl.ds(0, TAIL_R)],
                            out_hbm.at[pl.ds(N_ROW - TAIL_R, TAIL_R)])

    return k


def _make_lookup_kernel():
    mesh = plsc.VectorSubcoreMesh(core_axis_name="c", subcore_axis_name="s")

    @functools.partial(
        pl.kernel,
        mesh=mesh,
        compiler_params=pltpu.CompilerParams(needs_layout_passes=False, disable_bounds_checks=True),
        out_type=jax.ShapeDtypeStruct((L, D_OUT, B), jnp.float32),
        scratch_types=[
            pltpu.VMEM((L, BPW), jnp.int32),
            pltpu.VMEM((BPW,), jnp.int32),
            pltpu.VMEM((BPW,), jnp.int32),
            pltpu.VMEM((BPW, TAB_W), jnp.float32),
            pltpu.VMEM((N_FEATURE, BPW), jnp.float32),
            pltpu.VMEM((D_OUT, BPW), jnp.float32),
            pltpu.VMEM((24, 128), jnp.float32),
            pltpu.SMEM((N_FEATURE * D_FEATURE,), jnp.float32),
            pltpu.SMEM((D_FEATURE,), jnp.float32),
            pltpu.SemaphoreType.DMA,
        ],
    )
    def k(tok_hbm, feat_hbm, table_hbm, aux_hbm, out_hbm,
          tok_v, half_v, par_v, emb_v, feat_v, tile_v, aux_v,
          w_sm, b_sm, sem):
        cid = lax.axis_index("c")
        sid = lax.axis_index("s")
        wid = cid * NS + sid
        boff = pl.multiple_of(wid * BPW, BPW)

        pltpu.sync_copy(tok_hbm.at[:, pl.ds(boff, BPW)], tok_v)
        pltpu.sync_copy(aux_hbm, aux_v)
        for kf in range(N_FEATURE):
            for jh in range(D_FEATURE // 16):
                wv = aux_v[kf, pl.ds(jh * 16, 16)]
                for i in range(16):
                    w_sm[kf * D_FEATURE + jh * 16 + i] = wv[i]
        for jh in range(D_FEATURE // 16):
            bv = aux_v[N_FEATURE, pl.ds(jh * 16, 16)]
            for i in range(16):
                b_sm[jh * 16 + i] = bv[i]

        lane = lax.iota(jnp.int32, 16)

        def l_body(l, carry):
            for g in range(BPW // 16):
                t16 = tok_v[l, pl.ds(g * 16, 16)]
                half_v[pl.ds(g * 16, 16)] = t16 >> 1
                par_v[pl.ds(g * 16, 16)] = (t16 & 1) * D_EMBED
            pltpu.async_copy(table_hbm.at[half_v], emb_v, sem).wait()
            pltpu.sync_copy(feat_hbm.at[l, :, pl.ds(boff, BPW)], feat_v)

            for g in range(BPW // 16):
                fv = [feat_v[kf, pl.ds(g * 16, 16)] for kf in range(N_FEATURE)]

                def j_body(j, c2):
                    acc = fv[0] * w_sm[j] + b_sm[j]
                    for kf in range(1, N_FEATURE):
                        acc = acc + fv[kf] * w_sm[kf * D_FEATURE + j]
                    tile_v[D_EMBED + j, pl.ds(g * 16, 16)] = acc
                    return c2

                lax.fori_loop(0, D_FEATURE, j_body, 0)

            for g in range(BPW // 16):
                rows = lane + (g * 16)
                cols0 = par_v[pl.ds(g * 16, 16)]

                def c_body(c8, c2):
                    for dc in range(8):
                        c = c8 * 8 + dc
                        val = plsc.load_gather(emb_v, [rows, cols0 + c])
                        tile_v[c, pl.ds(g * 16, 16)] = val
                    return c2

                lax.fori_loop(0, D_EMBED // 8, c_body, 0)

            pltpu.sync_copy(tile_v, out_hbm.at[l, :, pl.ds(boff, BPW)])
            return carry

        lax.fori_loop(0, L, l_body, 0)

    return k


_relayout = _make_relayout_kernel()
_lookup = _make_lookup_kernel()


def kernel(tokens, features, embed_table, proj_W, proj_b):
    tok_t = tokens.astype(jnp.int32).T
    feat_t = features.transpose(1, 2, 0)
    tab_t = embed_table.T
    tail = embed_table[N_TOKEN - TAIL_T:, :]
    aux = jnp.zeros((24, 128), jnp.float32)
    aux = aux.at[:N_FEATURE, :D_FEATURE].set(proj_W.T)
    aux = aux.at[N_FEATURE, :D_FEATURE].set(proj_b)
    table2 = _relayout(tab_t, tail)
    out = _lookup(tok_t, feat_t, table2, aux)
    return out.transpose(2, 0, 1)


# speedup vs baseline: 2.1006x; 2.1000x over previous
import functools

import jax
import jax.numpy as jnp
from jax import lax
from jax.experimental import pallas as pl
from jax.experimental.pallas import tpu as pltpu
from jax.experimental.pallas import tpu_sc as plsc

B = 4096
L = 50
D_EMBED = 64
N_FEATURE = 16
D_FEATURE = 32
D_OUT = D_EMBED + D_FEATURE
N_TOKEN = 1000000
TAB_W = 128
N_ROW = N_TOKEN // 2
PAD_W = 129

NC, NS = 2, 16
NW = NC * NS
BPW = B // NW


def _make_lookup_kernel():
    mesh = plsc.VectorSubcoreMesh(core_axis_name="c", subcore_axis_name="s")

    @functools.partial(
        pl.kernel,
        mesh=mesh,
        compiler_params=pltpu.CompilerParams(needs_layout_passes=False,
                                             disable_bounds_checks=True),
        out_type=jax.ShapeDtypeStruct((L, D_OUT, B), jnp.float32),
        scratch_types=[
            pltpu.VMEM((L, BPW), jnp.int32),
            pltpu.VMEM((2, BPW), jnp.int32),
            pltpu.VMEM((2, BPW), jnp.int32),
            pltpu.VMEM((BPW, TAB_W), jnp.float32),
            pltpu.VMEM((BPW, TAB_W), jnp.float32),
            pltpu.VMEM((N_FEATURE, BPW), jnp.float32),
            pltpu.VMEM((N_FEATURE, BPW), jnp.float32),
            pltpu.VMEM((D_OUT, PAD_W), jnp.float32),
            pltpu.VMEM((24, 128), jnp.float32),
            pltpu.SMEM((N_FEATURE * D_FEATURE,), jnp.float32),
            pltpu.SMEM((D_FEATURE,), jnp.float32),
            pltpu.SemaphoreType.DMA,
            pltpu.SemaphoreType.DMA,
            pltpu.SemaphoreType.DMA,
        ],
    )
    def k(tok_hbm, feat_hbm, table_hbm, aux_hbm, out_hbm,
          tok_v, half_v, par_v, embA, embB, featA, featB, tile_v, aux_v,
          w_sm, b_sm, sem_a, sem_b, sem_o):
        cid = lax.axis_index("c")
        sid = lax.axis_index("s")
        wid = cid * NS + sid
        boff = pl.multiple_of(wid * BPW, BPW)

        pltpu.sync_copy(tok_hbm.at[:, pl.ds(boff, BPW)], tok_v)
        pltpu.sync_copy(aux_hbm, aux_v)
        for kf in range(N_FEATURE):
            for jh in range(D_FEATURE // 16):
                wv = aux_v[kf, pl.ds(jh * 16, 16)]
                for i in range(16):
                    w_sm[kf * D_FEATURE + jh * 16 + i] = wv[i]
        for jh in range(D_FEATURE // 16):
            bv = aux_v[N_FEATURE, pl.ds(jh * 16, 16)]
            for i in range(16):
                b_sm[jh * 16 + i] = bv[i]

        lane = lax.iota(jnp.int32, 16)

        def prep(l, slot):
            for g in range(BPW // 16):
                t16 = tok_v[l, pl.ds(g * 16, 16)]
                half_v[slot, pl.ds(g * 16, 16)] = t16 >> 1
                par_v[slot, pl.ds(g * 16, 16)] = (t16 & 1) * D_EMBED

        def start_in(l, slot, emb_v, feat_v, sem):
            pltpu.async_copy(table_hbm.at[half_v.at[slot]], emb_v, sem)
            pltpu.async_copy(feat_hbm.at[l, :, pl.ds(boff, BPW)], feat_v,
                             sem)

        def wait_in(emb_v, feat_v, sem):
            pltpu.make_async_copy(table_hbm.at[half_v.at[0]], emb_v,
                                  sem).wait()
            pltpu.make_async_copy(feat_hbm.at[0, :, pl.ds(boff, BPW)],
                                  feat_v, sem).wait()

        def compute(l, slot, emb_v, feat_v):
            for g in range(BPW // 16):
                fv = [feat_v[kf, pl.ds(g * 16, 16)] for kf in range(N_FEATURE)]

                def j_body(j, c2):
                    acc = fv[0] * w_sm[j] + b_sm[j]
                    for kf in range(1, N_FEATURE):
                        acc = acc + fv[kf] * w_sm[kf * D_FEATURE + j]
                    tile_v[D_EMBED + j, pl.ds(g * 16, 16)] = acc
                    return c2

                lax.fori_loop(0, D_FEATURE, j_body, 0)

            def g_body(g, c2):
                parv = par_v[slot, pl.ds(g * 16, 16)]
                for i in range(16):
                    b = g * 16 + i
                    pb = parv[i]
                    bvec = jnp.full_like(lane, b)
                    for c0 in range(0, D_EMBED, 16):
                        val = emb_v[b, pl.ds(pb + c0, 16)]
                        plsc.store_scatter(tile_v, [lane + c0, bvec], val)
                return c2

            lax.fori_loop(0, BPW // 16, g_body, 0)

            pltpu.sync_copy(tile_v.at[:, pl.ds(0, BPW)],
                            out_hbm.at[l, :, pl.ds(boff, BPW)])

        prep(0, 0)
        start_in(0, 0, embA, featA, sem_a)

        def pair_body(i, carry):
            lA = 2 * i
            prep(lA + 1, 1)
            start_in(lA + 1, 1, embB, featB, sem_b)
            wait_in(embA, featA, sem_a)
            compute(lA, 0, embA, featA)

            @pl.when(i < L // 2 - 1)
            def _():
                prep(lA + 2, 0)
                start_in(lA + 2, 0, embA, featA, sem_a)

            wait_in(embB, featB, sem_b)
            compute(lA + 1, 1, embB, featB)
            return carry

        lax.fori_loop(0, L // 2, pair_body, 0)

    return k


_lookup = _make_lookup_kernel()


def kernel(tokens, features, embed_table, proj_W, proj_b):
    tok_t = tokens.astype(jnp.int32).T
    feat_t = features.transpose(1, 2, 0)
    table2 = embed_table.reshape(N_ROW, TAB_W)
    aux = jnp.zeros((24, 128), jnp.float32)
    aux = aux.at[:N_FEATURE, :D_FEATURE].set(proj_W.T)
    aux = aux.at[N_FEATURE, :D_FEATURE].set(proj_b)
    out = _lookup(tok_t, feat_t, table2, aux)
    return out.transpose(2, 0, 1)
